# Initial kernel scaffold; baseline (speedup 1.0000x reference)
import jax
import jax.numpy as jnp
from jax.experimental import pallas as pl


def kernel(unlabel_feat, unlabel_prob, rot_unlabel_prob):
    raise NotImplementedError("write your pallas kernel here")


# speedup vs baseline: 10.9360x; 10.9360x over previous
import functools

import jax
import jax.numpy as jnp
from jax.experimental import pallas as pl
from jax.experimental.pallas import tpu as pltpu

B = 1024
D = 512
C = 1000
K = 5
BLK = 128
NBLK = B // BLK


def _keys_body(feat_ref, keys_ref):
    x = feat_ref[:]
    iota = jax.lax.broadcasted_iota(jnp.int32, (B, D), 1)
    idxs = []
    for _ in range(K):
        m = jnp.max(x, axis=1, keepdims=True)
        idx = jnp.min(jnp.where(x == m, iota, D), axis=1)
        idxs.append(idx)
        x = jnp.where(iota == idx[:, None], -jnp.inf, x)
    a = (idxs[0] * D + idxs[1]) * D + idxs[2]
    b = idxs[3] * D + idxs[4]
    zeros = jnp.zeros((6, B), jnp.int32)
    keys_ref[:] = jnp.concatenate([a[None, :], b[None, :], zeros], axis=0)


def _loss_body(p_ref, r_ref, keys_ref, out_ref):
    i = pl.program_id(0)
    prod = jax.lax.dot_general(
        p_ref[:], r_ref[:], (((1,), (1,)), ((), ())),
        preferred_element_type=jnp.float32)
    ka = keys_ref[0:1, :]
    kb = keys_ref[1:2, :]
    my_a = jnp.reshape(keys_ref[0:1, pl.ds(i * BLK, BLK)], (BLK, 1))
    my_b = jnp.reshape(keys_ref[1:2, pl.ds(i * BLK, BLK)], (BLK, 1))
    sim = ((my_a == ka) & (my_b == kb)).astype(jnp.float32)
    log_p = jnp.maximum(jnp.log(prod), -100.0)
    log_1mp = jnp.maximum(jnp.log(1.0 - prod), -100.0)
    loss = sim * (log_1mp - log_p) - log_1mp
    partial = jnp.full((1, 1), 1.0 / (B * B)) * jnp.sum(loss)

    @pl.when(i == 0)
    def _():
        out_ref[:, :] = jnp.zeros((1, 1), jnp.float32)

    out_ref[:, :] += partial


@jax.jit
def kernel(unlabel_feat, unlabel_prob, rot_unlabel_prob):
    keys = pl.pallas_call(
        _keys_body,
        out_shape=jax.ShapeDtypeStruct((8, B), jnp.int32),
    )(unlabel_feat)

    out = pl.pallas_call(
        _loss_body,
        grid=(NBLK,),
        in_specs=[
            pl.BlockSpec((BLK, C), lambda i: (i, 0)),
            pl.BlockSpec((B, C), lambda i: (0, 0)),
            pl.BlockSpec((8, B), lambda i: (0, 0)),
        ],
        out_specs=pl.BlockSpec((1, 1), lambda i: (0, 0)),
        out_shape=jax.ShapeDtypeStruct((1, 1), jnp.float32),
    )(unlabel_prob, rot_unlabel_prob, keys)
    return out[0, 0]


# speedup vs baseline: 11.6281x; 1.0633x over previous
import jax
import jax.numpy as jnp
from jax.experimental import pallas as pl
from jax.experimental.pallas import tpu as pltpu

B = 1024
D = 512
C = 1000
K = 5
BLK = 128
NBLK = B // BLK


def _body(feat_ref, p_ref, r_ref, out_ref, keys_ref):
    i = pl.program_id(0)

    @pl.when(i == 0)
    def _():
        x = feat_ref[:]
        iota = jax.lax.broadcasted_iota(jnp.int32, (B, D), 1)
        idxs = []
        for _ in range(K):
            m = jnp.max(x, axis=1, keepdims=True)
            idx = jnp.min(jnp.where(x == m, iota, D), axis=1)
            idxs.append(idx)
            x = jnp.where(iota == idx[:, None], -jnp.inf, x)
        a = (idxs[0] * D + idxs[1]) * D + idxs[2]
        b = idxs[3] * D + idxs[4]
        keys_ref[:] = jnp.concatenate(
            [a[None, :], b[None, :], jnp.zeros((6, B), jnp.int32)], axis=0)

    prod = jax.lax.dot_general(
        p_ref[:], r_ref[:], (((1,), (1,)), ((), ())),
        preferred_element_type=jnp.float32)
    ka = keys_ref[0:1, :]
    kb = keys_ref[1:2, :]
    my_a = jnp.reshape(keys_ref[0:1, pl.ds(i * BLK, BLK)], (BLK, 1))
    my_b = jnp.reshape(keys_ref[1:2, pl.ds(i * BLK, BLK)], (BLK, 1))
    sim = ((my_a == ka) & (my_b == kb)).astype(jnp.float32)
    log_p = jnp.maximum(jnp.log(prod), -100.0)
    log_1mp = jnp.maximum(jnp.log(1.0 - prod), -100.0)
    loss = sim * (log_1mp - log_p) - log_1mp
    partial = jnp.full((1, 1), 1.0 / (B * B)) * jnp.sum(loss)

    @pl.when(i == 0)
    def _():
        out_ref[:, :] = jnp.zeros((1, 1), jnp.float32)

    out_ref[:, :] += partial


@jax.jit
def kernel(unlabel_feat, unlabel_prob, rot_unlabel_prob):
    out = pl.pallas_call(
        _body,
        grid=(NBLK,),
        in_specs=[
            pl.BlockSpec((B, D), lambda i: (0, 0)),
            pl.BlockSpec((BLK, C), lambda i: (i, 0)),
            pl.BlockSpec((B, C), lambda i: (0, 0)),
        ],
        out_specs=pl.BlockSpec((1, 1), lambda i: (0, 0)),
        out_shape=jax.ShapeDtypeStruct((1, 1), jnp.float32),
        scratch_shapes=[pltpu.VMEM((8, B), jnp.int32)],
    )(unlabel_feat, unlabel_prob, rot_unlabel_prob)
    return out[0, 0]


# speedup vs baseline: 13.9344x; 1.1983x over previous
import jax
import jax.numpy as jnp
from jax.experimental import pallas as pl
from jax.experimental.pallas import tpu as pltpu

B = 1024
D = 512
C = 1000
K = 5
BLK = 128
NBLK = B // BLK


def _body(feat_hbm, p_hbm, r_hbm, out_ref,
          feat_v, p_v, r_v, keys_v, sem_f, sem_p, sem_r):
    cp_f = pltpu.make_async_copy(feat_hbm, feat_v, sem_f)
    cp_p = pltpu.make_async_copy(p_hbm, p_v, sem_p)
    cp_r = pltpu.make_async_copy(r_hbm, r_v, sem_r)
    cp_f.start()
    cp_p.start()
    cp_r.start()

    cp_f.wait()
    x = feat_v[:]
    iota = jax.lax.broadcasted_iota(jnp.int32, (B, D), 1)
    idxs = []
    for _ in range(K):
        m = jnp.max(x, axis=1, keepdims=True)
        idx = jnp.min(jnp.where(x == m, iota, D), axis=1)
        idxs.append(idx)
        x = jnp.where(iota == idx[:, None], -jnp.inf, x)
    a = (idxs[0] * D + idxs[1]) * D + idxs[2]
    b = idxs[3] * D + idxs[4]
    keys_v[:] = jnp.concatenate(
        [a[None, :], b[None, :], jnp.zeros((6, B), jnp.int32)], axis=0)

    cp_p.wait()
    cp_r.wait()

    ka = keys_v[0:1, :]
    kb = keys_v[1:2, :]
    r_all = r_v[:]
    acc = jnp.zeros((1, 1), jnp.float32)
    for k in range(NBLK):
        prod = jax.lax.dot_general(
            p_v[k * BLK:(k + 1) * BLK, :], r_all,
            (((1,), (1,)), ((), ())),
            preferred_element_type=jnp.float32)
        my_a = jnp.reshape(keys_v[0:1, k * BLK:(k + 1) * BLK], (BLK, 1))
        my_b = jnp.reshape(keys_v[1:2, k * BLK:(k + 1) * BLK], (BLK, 1))
        sim = ((my_a == ka) & (my_b == kb)).astype(jnp.float32)
        log_p = jnp.maximum(jnp.log(prod), -100.0)
        log_1mp = jnp.maximum(jnp.log(1.0 - prod), -100.0)
        loss = sim * (log_1mp - log_p) - log_1mp
        acc += jnp.full((1, 1), 1.0 / (B * B)) * jnp.sum(loss)
    out_ref[:, :] = acc


@jax.jit
def kernel(unlabel_feat, unlabel_prob, rot_unlabel_prob):
    out = pl.pallas_call(
        _body,
        grid=(1,),
        in_specs=[
            pl.BlockSpec(memory_space=pl.ANY),
            pl.BlockSpec(memory_space=pl.ANY),
            pl.BlockSpec(memory_space=pl.ANY),
        ],
        out_specs=pl.BlockSpec((1, 1), lambda i: (0, 0)),
        out_shape=jax.ShapeDtypeStruct((1, 1), jnp.float32),
        scratch_shapes=[
            pltpu.VMEM((B, D), jnp.float32),
            pltpu.VMEM((B, C), jnp.float32),
            pltpu.VMEM((B, C), jnp.float32),
            pltpu.VMEM((8, B), jnp.int32),
            pltpu.SemaphoreType.DMA,
            pltpu.SemaphoreType.DMA,
            pltpu.SemaphoreType.DMA,
        ],
    )(unlabel_feat, unlabel_prob, rot_unlabel_prob)
    return out[0, 0]
